# baseline (speedup 1.0000x reference)
import functools

import jax
import jax.numpy as jnp
from jax import lax
from jax.experimental import pallas as pl
from jax.experimental.pallas import tpu as pltpu
from jax.experimental.pallas import tpu_sc as plsc


def kernel(x):
    B, C, H, W = x.shape
    R = B * C
    xf = x.reshape(R, H, W)
    fours = jnp.full((1, H, W), 4.0, x.dtype)

    info = plsc.get_sparse_core_info()
    NW = info.num_cores * info.num_subcores
    rpw = R // NW

    mesh = plsc.VectorSubcoreMesh(core_axis_name="c", subcore_axis_name="s")

    @functools.partial(
        pl.kernel,
        mesh=mesh,
        out_type=jax.ShapeDtypeStruct((R, H, W), x.dtype),
        scratch_types=[pltpu.SemaphoreType.DMA],
    )
    def sc_copy(x_hbm, fours_hbm, out_hbm, sem):
        wid = lax.axis_index("s") * info.num_cores + lax.axis_index("c")
        base = wid * rpw
        is_even = (wid % 2) == 0

        @pl.when(is_even)
        def _():
            c0 = pltpu.async_copy(
                x_hbm.at[pl.ds(base, 1)], out_hbm.at[pl.ds(base, 1)], sem)
            c1 = pltpu.async_copy(
                fours_hbm.at[pl.ds(0, 1)], out_hbm.at[pl.ds(base + 1, 1)], sem)
            c2 = pltpu.async_copy(
                x_hbm.at[pl.ds(base + 2, rpw - 2)],
                out_hbm.at[pl.ds(base + 2, rpw - 2)], sem)
            c0.wait()
            c1.wait()
            c2.wait()

        @pl.when(jnp.logical_not(is_even))
        def _():
            pltpu.async_copy(
                x_hbm.at[pl.ds(base, rpw)], out_hbm.at[pl.ds(base, rpw)],
                sem).wait()

    return sc_copy(xf, fours).reshape(B, C, H, W)


# speedup vs baseline: 1.0186x; 1.0186x over previous
import jax
import jax.numpy as jnp
from jax.experimental import pallas as pl
from jax.experimental.pallas import tpu as pltpu


def kernel(x):
    B, C, H, W = x.shape
    R = B * C
    xf = x.reshape(R, H, W)

    def body(x_hbm, o_hbm, fours, sem):
        fours[...] = jnp.full((1, H, W), 4.0, x.dtype)
        copies = []
        copies.append(pltpu.make_async_copy(
            x_hbm.at[pl.ds(0, 1)], o_hbm.at[pl.ds(0, 1)], sem))
        for k in range(B - 1):
            s = 64 * k + 2
            copies.append(pltpu.make_async_copy(
                x_hbm.at[pl.ds(s, 63)], o_hbm.at[pl.ds(s, 63)], sem))
        s = 64 * (B - 1) + 2
        copies.append(pltpu.make_async_copy(
            x_hbm.at[pl.ds(s, 62)], o_hbm.at[pl.ds(s, 62)], sem))
        for k in range(B):
            copies.append(pltpu.make_async_copy(
                fours, o_hbm.at[pl.ds(64 * k + 1, 1)], sem))
        for c in copies:
            c.start()
        for c in copies:
            c.wait()

    out = pl.pallas_call(
        body,
        in_specs=[pl.BlockSpec(memory_space=pl.ANY)],
        out_specs=pl.BlockSpec(memory_space=pl.ANY),
        out_shape=jax.ShapeDtypeStruct((R, H, W), x.dtype),
        scratch_shapes=[pltpu.VMEM((1, H, W), x.dtype), pltpu.SemaphoreType.DMA],
    )(xf)
    return out.reshape(B, C, H, W)


# speedup vs baseline: 36.4983x; 35.8307x over previous
import functools

import jax
import jax.numpy as jnp
from jax import lax
from jax.experimental import pallas as pl
from jax.experimental.pallas import tpu as pltpu
from jax.experimental.pallas import tpu_sc as plsc


def kernel(x):
    B, C, H, W = x.shape
    R = B * C
    HH = H // 2
    xf = x.reshape(R, H, W)
    fours = jnp.full((1, HH, W), 4.0, x.dtype)

    info = plsc.get_sparse_core_info()
    NW = info.num_cores * info.num_subcores
    rpw = R // NW
    U = 2 * rpw

    mesh = plsc.VectorSubcoreMesh(core_axis_name="c", subcore_axis_name="s")

    @functools.partial(
        pl.kernel,
        mesh=mesh,
        out_type=jax.ShapeDtypeStruct((R, H, W), x.dtype),
        scratch_types=[
            pltpu.VMEM((2, HH, W), x.dtype),
            pltpu.VMEM((1, HH, W), x.dtype),
            pltpu.SemaphoreType.DMA,
            pltpu.SemaphoreType.DMA,
        ],
    )
    def sc_copy(x_hbm, fours_hbm, out_hbm, buf, fv, in_sem, out_sem):
        wid = lax.axis_index("s") * info.num_cores + lax.axis_index("c")
        base = wid * rpw
        is_even = (wid % 2) == 0
        pltpu.sync_copy(fours_hbm, fv)

        def load(u, slot):
            row = base + u // 2
            return pltpu.make_async_copy(
                x_hbm.at[pl.ds(row, 1), pl.ds((u % 2) * HH, HH)],
                buf.at[pl.ds(slot, 1)], in_sem)

        def store(u, slot):
            row = base + u // 2
            dst = out_hbm.at[pl.ds(row, 1), pl.ds((u % 2) * HH, HH)]
            desc = pltpu.make_async_copy(buf.at[pl.ds(slot, 1)], dst, out_sem)
            if u // 2 == 1:
                fdesc = pltpu.make_async_copy(fv, dst, out_sem)

                @pl.when(is_even)
                def _():
                    fdesc.start()

                @pl.when(jnp.logical_not(is_even))
                def _():
                    desc.start()
            else:
                desc.start()
            return desc

        loads = {0: load(0, 0)}
        loads[0].start()
        stores = {}
        for u in range(U):
            slot = u % 2
            loads[u].wait()
            if u >= 1:
                stores[u - 1].wait()
            if u + 1 < U:
                loads[u + 1] = load(u + 1, 1 - slot)
                loads[u + 1].start()
            stores[u] = store(u, slot)
        stores[U - 1].wait()

    return sc_copy(xf, fours).reshape(B, C, H, W)


# speedup vs baseline: 50.2548x; 1.3769x over previous
import jax
import jax.numpy as jnp
from jax.experimental import pallas as pl
from jax.experimental.pallas import tpu as pltpu


def kernel(x):
    B, C, H, W = x.shape
    R = B * C
    CH = 32
    NS = 8
    NCH = R // CH
    xf = x.reshape(R, H, W)

    def body(x_hbm, o_hbm, buf, lsem, ssem):
        for s in range(0, NS, 2):
            buf[pl.ds(CH * s + 1, 1)] = jnp.full((1, H, W), 4.0, x.dtype)

        def loads(c):
            s = c % NS
            b0, r0 = CH * s, CH * c
            if c % 2 == 0:
                return [
                    pltpu.make_async_copy(
                        x_hbm.at[pl.ds(r0, 1)], buf.at[pl.ds(b0, 1)],
                        lsem.at[s]),
                    pltpu.make_async_copy(
                        x_hbm.at[pl.ds(r0 + 2, CH - 2)],
                        buf.at[pl.ds(b0 + 2, CH - 2)], lsem.at[s]),
                ]
            return [pltpu.make_async_copy(
                x_hbm.at[pl.ds(r0, CH)], buf.at[pl.ds(b0, CH)], lsem.at[s])]

        def store(c):
            s = c % NS
            return pltpu.make_async_copy(
                buf.at[pl.ds(CH * s, CH)], o_hbm.at[pl.ds(CH * c, CH)],
                ssem.at[s])

        pending = {}
        for c in range(NS):
            pending[c] = loads(c)
            for d in pending[c]:
                d.start()
        stores = {}
        for c in range(NCH):
            for d in pending.pop(c):
                d.wait()
            stores[c] = store(c)
            stores[c].start()
            if c + NS < NCH:
                stores[c].wait()
                pending[c + NS] = loads(c + NS)
                for d in pending[c + NS]:
                    d.start()
        for c in range(NCH - NS, NCH):
            stores[c].wait()

    out = pl.pallas_call(
        body,
        in_specs=[pl.BlockSpec(memory_space=pl.ANY)],
        out_specs=pl.BlockSpec(memory_space=pl.ANY),
        out_shape=jax.ShapeDtypeStruct((R, H, W), x.dtype),
        scratch_shapes=[
            pltpu.VMEM((NS * CH, H, W), x.dtype),
            pltpu.SemaphoreType.DMA((NS,)),
            pltpu.SemaphoreType.DMA((NS,)),
        ],
        compiler_params=pltpu.CompilerParams(
            vmem_limit_bytes=100 * 1024 * 1024),
    )(xf)
    return out.reshape(B, C, H, W)
